# baseline (speedup 1.0000x reference)
import dataclasses
import functools

import jax
import jax.numpy as jnp
from jax import lax
from jax.experimental import pallas as pl
from jax.experimental.pallas import tpu as pltpu
from jax.experimental.pallas import tpu_sc as plsc

NC = 2
NS = 16
L = 16
NW = NC * NS

BATCH = 16384
D = 32
BPW = BATCH // NW
NBLK = BPW // L


def _mf_body(uidx_hbm, iidx_hbm, w_hbm, h_hbm, out_hbm,
             uidx_v, iidx_v, u_v, v_v, out_v, sem_u, sem_v):
    wid = lax.axis_index("c") * NS + lax.axis_index("s")
    base = wid * BPW

    pltpu.sync_copy(uidx_hbm.at[pl.ds(base, BPW)], uidx_v)
    pltpu.sync_copy(iidx_hbm.at[pl.ds(base, BPW)], iidx_v)

    cu = pltpu.async_copy(w_hbm.at[uidx_v], u_v, sem_u)
    cv = pltpu.async_copy(h_hbm.at[iidx_v], v_v, sem_v)
    cu.wait()
    cv.wait()

    iota = lax.iota(jnp.int32, L)

    @pl.loop(0, BPW, step=L)
    def _(b):
        rows = b + iota
        acc = jnp.zeros((L,), jnp.float32)
        for j in range(D):
            cols = jnp.bitwise_and(iota + j, D - 1)
            u = plsc.load_gather(u_v, [rows, cols])
            v = plsc.load_gather(v_v, [rows, cols])
            acc = acc + u * v
        out_v[pl.ds(b, L)] = 1.0 / (1.0 + jnp.exp(-acc))

    pltpu.sync_copy(out_v, out_hbm.at[pl.ds(base, BPW)])


def _compiler_params():
    cp = pltpu.CompilerParams()
    fields = pltpu.CompilerParams.__dataclass_fields__
    if "needs_layout_passes" in fields:
        cp = dataclasses.replace(cp, needs_layout_passes=False)
    if "use_tc_tiling_on_sc" in fields:
        cp = dataclasses.replace(cp, use_tc_tiling_on_sc=False)
    return cp


def kernel(user_idx, item_idx, W, H):
    mesh = plsc.VectorSubcoreMesh(core_axis_name="c", subcore_axis_name="s")
    mf = functools.partial(
        pl.kernel,
        out_type=jax.ShapeDtypeStruct((BATCH,), jnp.float32),
        mesh=mesh,
        scratch_types=[
            pltpu.VMEM((BPW,), jnp.int32),
            pltpu.VMEM((BPW,), jnp.int32),
            pltpu.VMEM((BPW, D), jnp.float32),
            pltpu.VMEM((BPW, D), jnp.float32),
            pltpu.VMEM((BPW,), jnp.float32),
            pltpu.SemaphoreType.DMA,
            pltpu.SemaphoreType.DMA,
        ],
        compiler_params=_compiler_params(),
    )(_mf_body)
    return mf(user_idx.astype(jnp.int32), item_idx.astype(jnp.int32), W, H)


# speedup vs baseline: 1.5001x; 1.5001x over previous
import dataclasses
import functools

import jax
import jax.numpy as jnp
from jax import lax
from jax.experimental import pallas as pl
from jax.experimental.pallas import tpu as pltpu
from jax.experimental.pallas import tpu_sc as plsc

NC = 2
NS = 16
L = 16
NW = NC * NS

BATCH = 16384
D = 32
BPW = BATCH // NW
CH = 256
NCHUNK = BPW // CH


def _mf_body(uidx_hbm, iidx_hbm, w_hbm, h_hbm, out_hbm,
             uidx_v, iidx_v, u_v, v_v, out_v, sem_u, sem_v):
    wid = lax.axis_index("c") * NS + lax.axis_index("s")
    base = wid * BPW

    pltpu.sync_copy(uidx_hbm.at[pl.ds(base, BPW)], uidx_v)
    pltpu.sync_copy(iidx_hbm.at[pl.ds(base, BPW)], iidx_v)

    iota = lax.iota(jnp.int32, L)

    @pl.loop(0, NCHUNK)
    def _(c):
        co = c * CH

        @pl.loop(0, CH, step=L)
        def _(i0):
            uvec = uidx_v[pl.ds(co + i0, L)]
            for lane in range(L):
                pltpu.make_async_copy(
                    w_hbm.at[pl.ds(uvec[lane], 1), :],
                    u_v.at[pl.ds(i0 + lane, 1), :], sem_u
                ).start()

        @pl.loop(0, CH, step=L)
        def _(i0):
            ivec = iidx_v[pl.ds(co + i0, L)]
            for lane in range(L):
                pltpu.make_async_copy(
                    h_hbm.at[pl.ds(ivec[lane], 1), :],
                    v_v.at[pl.ds(i0 + lane, 1), :], sem_v
                ).start()

        pltpu.make_async_copy(w_hbm.at[pl.ds(0, CH), :], u_v, sem_u).wait()
        pltpu.make_async_copy(h_hbm.at[pl.ds(0, CH), :], v_v, sem_v).wait()

        @pl.loop(0, CH, step=L)
        def _(b):
            rows = b + iota
            acc = jnp.zeros((L,), jnp.float32)
            for j in range(D):
                cols = jnp.bitwise_and(iota + j, D - 1)
                u = plsc.load_gather(u_v, [rows, cols])
                v = plsc.load_gather(v_v, [rows, cols])
                acc = acc + u * v
            out_v[pl.ds(co + b, L)] = 1.0 / (1.0 + jnp.exp(-acc))

    pltpu.sync_copy(out_v, out_hbm.at[pl.ds(base, BPW)])


def _compiler_params():
    cp = pltpu.CompilerParams()
    fields = pltpu.CompilerParams.__dataclass_fields__
    if "needs_layout_passes" in fields:
        cp = dataclasses.replace(cp, needs_layout_passes=False)
    return cp


def kernel(user_idx, item_idx, W, H):
    mesh = plsc.VectorSubcoreMesh(core_axis_name="c", subcore_axis_name="s")
    mf = functools.partial(
        pl.kernel,
        out_type=jax.ShapeDtypeStruct((BATCH,), jnp.float32),
        mesh=mesh,
        scratch_types=[
            pltpu.VMEM((BPW,), jnp.int32),
            pltpu.VMEM((BPW,), jnp.int32),
            pltpu.VMEM((CH, D), jnp.float32),
            pltpu.VMEM((CH, D), jnp.float32),
            pltpu.VMEM((BPW,), jnp.float32),
            pltpu.SemaphoreType.DMA,
            pltpu.SemaphoreType.DMA,
        ],
        compiler_params=_compiler_params(),
    )(_mf_body)
    return mf(user_idx.astype(jnp.int32), item_idx.astype(jnp.int32), W, H)
